# baseline (speedup 1.0000x reference)
import jax
import jax.numpy as jnp
from jax import lax
from jax.experimental import pallas as pl
from jax.experimental.pallas import tpu as pltpu
from jax.experimental.pallas import tpu_sc as plsc

N = 10000
E = 320000
C = 128
R = 8
NT = 4
NB = 4

NC = 2
NS = 16
EPT = E // (NC * NS)
RPT = N // NS
DRN = 632
CNT_PAD = 81920
TPB = CNT_PAD // NS
E_CHK = 80
N_CHK = EPT // E_CHK
E_BLK = 2000
RB = 2560

_f32 = jnp.float32
_i32 = jnp.int32
_sc_params = pltpu.CompilerParams(needs_layout_passes=False)


def _mesh():
    return plsc.VectorSubcoreMesh(core_axis_name="c", subcore_axis_name="s")


def _stage_a_body(x_ref, comp_ref, basis_ref, root_ref, h_ref, xr_ref):
    xb = x_ref[...]
    comp = comp_ref[...]
    basis = basis_ref[...]
    for r in range(R):
        w = comp[r, 0] * basis[0]
        for b in range(1, NB):
            w = w + comp[r, b] * basis[b]
        h_ref[:, r * C:(r + 1) * C] = jnp.dot(
            xb, w, preferred_element_type=_f32)
    xr_ref[...] = jnp.dot(xb, root_ref[...], preferred_element_type=_f32)


def _stage_a(x, comp, basis, root):
    blk = 2000
    return pl.pallas_call(
        _stage_a_body,
        grid=(N // blk,),
        in_specs=[
            pl.BlockSpec((blk, C), lambda i: (i, 0)),
            pl.BlockSpec((R, NB), lambda i: (0, 0)),
            pl.BlockSpec((NB, C, C), lambda i: (0, 0, 0)),
            pl.BlockSpec((C, C), lambda i: (0, 0)),
        ],
        out_specs=[
            pl.BlockSpec((blk, R * C), lambda i: (i, 0)),
            pl.BlockSpec((blk, C), lambda i: (i, 0)),
        ],
        out_shape=[
            jax.ShapeDtypeStruct((N, R * C), _f32),
            jax.ShapeDtypeStruct((N, C), _f32),
        ],
    )(x, comp, basis, root)


CROWS = CNT_PAD // C
CRPT = CROWS // NS
CCHK = 128


def _stage_b_body(zeros_hbm, src_hbm, dst_hbm, et_hbm,
                  gidx_hbm, escale_hbm,
                  cnt_loc, srcb, dstb, etb, sclb, rowidx, spcnt):
    cid = lax.axis_index("c")
    sid = lax.axis_index("s")
    base = cid * (E // NC) + sid * EPT

    pltpu.sync_copy(zeros_hbm, cnt_loc)
    pltpu.sync_copy(zeros_hbm.at[pl.ds(sid * CRPT, CRPT)],
                    spcnt.at[pl.ds(sid * CRPT, CRPT)])

    for half in range(NC):
        hbase = half * (E // NC) + sid * EPT
        pltpu.sync_copy(dst_hbm.at[pl.ds(hbase, EPT)], dstb)
        pltpu.sync_copy(et_hbm.at[pl.ds(hbase, EPT)], etb)

        def count_body(g, _):
            d16 = dstb[pl.ds(16 * g, 16)]
            t16 = etb[pl.ds(16 * g, 16)]
            cidx = d16 * R + t16
            cnts, msk = plsc.scan_count(cidx)
            plsc.addupdate_scatter(
                cnt_loc,
                [lax.shift_right_logical(cidx, 7), cidx & (C - 1)],
                cnts.astype(_f32), mask=msk)
            return 0
        lax.fori_loop(0, EPT // 16, count_body, 0, unroll=4)

    plsc.subcore_barrier()

    for k in range(CROWS // CCHK):
        def idx_body(i, _):
            rowidx[pl.ds(16 * i, 16)] = (
                lax.iota(_i32, 16) + (k * CCHK + 16 * i))
            return 0
        lax.fori_loop(0, CCHK // 16, idx_body, 0)
        pltpu.sync_copy(cnt_loc.at[pl.ds(k * CCHK, CCHK)],
                        spcnt.at[rowidx], add=True)

    plsc.subcore_barrier()

    pltpu.sync_copy(spcnt.at[pl.ds(sid * CRPT, CRPT)],
                    cnt_loc.at[pl.ds(0, CRPT)])
    for e in range(CRPT):
        for c in range(C // 16):
            cnt_loc[e, pl.ds(16 * c, 16)] = (
                1.0 / jnp.maximum(cnt_loc[e, pl.ds(16 * c, 16)], 1.0))
    pltpu.sync_copy(cnt_loc.at[pl.ds(0, CRPT)],
                    spcnt.at[pl.ds(sid * CRPT, CRPT)])

    plsc.subcore_barrier()
    pltpu.sync_copy(spcnt, cnt_loc)

    pltpu.sync_copy(src_hbm.at[pl.ds(base, EPT)], srcb)
    pltpu.sync_copy(dst_hbm.at[pl.ds(base, EPT)], dstb)
    pltpu.sync_copy(et_hbm.at[pl.ds(base, EPT)], etb)

    def gs_body(g, _):
        s16 = srcb[pl.ds(16 * g, 16)]
        d16 = dstb[pl.ds(16 * g, 16)]
        t16 = etb[pl.ds(16 * g, 16)]
        cidx = d16 * R + t16
        srcb[pl.ds(16 * g, 16)] = s16 * R + t16
        sclb[pl.ds(16 * g, 16)] = plsc.load_gather(
            cnt_loc,
            [lax.shift_right_logical(cidx, 7), cidx & (C - 1)])
        return 0
    lax.fori_loop(0, EPT // 16, gs_body, 0, unroll=4)

    pltpu.sync_copy(srcb, gidx_hbm.at[pl.ds(base, EPT)])
    pltpu.sync_copy(sclb, escale_hbm.at[pl.ds(base, EPT)])


def _stage_b(src, dst, et):
    zeros = jnp.zeros((CROWS, C), _f32)
    return pl.kernel(
        _stage_b_body,
        out_type=(jax.ShapeDtypeStruct((E,), _i32),
                  jax.ShapeDtypeStruct((E,), _f32)),
        mesh=_mesh(),
        compiler_params=_sc_params,
        scratch_types=[
            pltpu.VMEM((CROWS, C), _f32),
            pltpu.VMEM((EPT,), _i32),
            pltpu.VMEM((EPT,), _i32),
            pltpu.VMEM((EPT,), _i32),
            pltpu.VMEM((EPT,), _f32),
            pltpu.VMEM((CCHK,), _i32),
            pltpu.VMEM_SHARED((CROWS, C), _f32),
        ],
    )(zeros, src, dst, et)


def _stage_c_body(hflat_hbm, gidx_hbm, dst_hbm, escale_hbm, out2_hbm,
                  gidxb, dstb, sclb, dstw, rows0, rows1, sem0, sem1, outsp):
    cid = lax.axis_index("c")
    sid = lax.axis_index("s")
    base = cid * (E // NC) + sid * EPT
    rows_bufs = (rows0, rows1)
    sems = (sem0, sem1)

    for e in range(E_CHK):
        for c in range(C // 16):
            rows0[e, pl.ds(16 * c, 16)] = jnp.zeros((16,), _f32)

    def zero_out(k, _):
        pltpu.sync_copy(rows0, outsp.at[pl.ds(sid * RPT + k * E_CHK, E_CHK)])
        return 0
    lax.fori_loop(0, RPT // E_CHK, zero_out, 0)
    pltpu.sync_copy(rows0.at[pl.ds(0, RPT % E_CHK)],
                    outsp.at[pl.ds(sid * RPT + (RPT // E_CHK) * E_CHK,
                                   RPT % E_CHK)])

    pltpu.sync_copy(gidx_hbm.at[pl.ds(base, EPT)], gidxb)
    pltpu.sync_copy(dst_hbm.at[pl.ds(base, EPT)], dstb)
    pltpu.sync_copy(escale_hbm.at[pl.ds(base, EPT)], sclb)

    plsc.subcore_barrier()

    pltpu.async_copy(hflat_hbm.at[gidxb.at[pl.ds(0, E_CHK)]],
                     rows0, sem0)

    def process(ci, rows, sem):
        off = ci * E_CHK
        for j in range(E_CHK // 16):
            dstw[pl.ds(16 * j, 16)] = dstb[pl.ds(off + 16 * j, 16)]
        pltpu.make_async_copy(hflat_hbm.at[gidxb.at[pl.ds(off, E_CHK)]],
                              rows, sem).wait()
        for j in range(E_CHK // 16):
            sg = sclb[pl.ds(off + 16 * j, 16)]
            for el in range(16):
                e = 16 * j + el
                sv = sg[el]
                for c in range(C // 16):
                    rows[e, pl.ds(16 * c, 16)] = (
                        rows[e, pl.ds(16 * c, 16)] * sv)
        pltpu.sync_copy(rows, outsp.at[dstw], add=True)

    def chunk_body(cp, _):
        for par in range(2):
            ci = 2 * cp + par
            nxt = ci + 1
            @pl.when(nxt < N_CHK)
            def _fire():
                pltpu.async_copy(
                    hflat_hbm.at[gidxb.at[pl.ds(nxt * E_CHK, E_CHK)]],
                    rows_bufs[(par + 1) % 2], sems[(par + 1) % 2])
            process(ci, rows_bufs[par], sems[par])
        return 0
    lax.fori_loop(0, N_CHK // 2, chunk_body, 0)
    process(N_CHK - 1, rows0, sem0)

    plsc.subcore_barrier()

    @pl.when(sid < NS - 1)
    def _drain_main():
        pltpu.sync_copy(outsp.at[pl.ds(sid * DRN, DRN)],
                        out2_hbm.at[cid, pl.ds(sid * DRN, DRN)])

    @pl.when(sid == NS - 1)
    def _drain_tail():
        pltpu.sync_copy(outsp.at[pl.ds((NS - 1) * DRN, N - (NS - 1) * DRN)],
                        out2_hbm.at[cid, pl.ds((NS - 1) * DRN,
                                               N - (NS - 1) * DRN)])


def _stage_c(hflat, gidx, dst, escale):
    return pl.kernel(
        _stage_c_body,
        out_type=jax.ShapeDtypeStruct((NC, N, C), _f32),
        mesh=_mesh(),
        compiler_params=_sc_params,
        scratch_types=[
            pltpu.VMEM((EPT,), _i32),
            pltpu.VMEM((EPT,), _i32),
            pltpu.VMEM((EPT,), _f32),
            pltpu.VMEM((E_CHK,), _i32),
            pltpu.VMEM((E_CHK, C), _f32),
            pltpu.VMEM((E_CHK, C), _f32),
            pltpu.SemaphoreType.DMA,
            pltpu.SemaphoreType.DMA,
            pltpu.VMEM_SHARED((N, C), _f32),
        ],
    )(hflat, gidx, dst, escale)


def _stage_d_body(p_ref, xr_ref, nt_ref, bias_ref, hw_ref, hb_ref, out_ref):
    h = p_ref[0] + p_ref[1] + xr_ref[...] + bias_ref[...]
    h = jnp.maximum(h, 0.0)
    nt = nt_ref[...]
    acc = jnp.zeros_like(xr_ref[...])
    for t in range(NT):
        m = (nt == t).astype(_f32)
        acc = acc + jnp.dot(h * m, hw_ref[t], preferred_element_type=_f32)
        acc = acc + m * hb_ref[t:t + 1, :]
    out_ref[...] = acc


def _stage_d(parts, xroot, nt2, bias2, het_w, het_b):
    blk = 2000
    return pl.pallas_call(
        _stage_d_body,
        grid=(N // blk,),
        in_specs=[
            pl.BlockSpec((NC, blk, C), lambda i: (0, i, 0)),
            pl.BlockSpec((blk, C), lambda i: (i, 0)),
            pl.BlockSpec((blk, 1), lambda i: (i, 0)),
            pl.BlockSpec((1, C), lambda i: (0, 0)),
            pl.BlockSpec((NT, C, C), lambda i: (0, 0, 0)),
            pl.BlockSpec((NT, C), lambda i: (0, 0)),
        ],
        out_specs=pl.BlockSpec((blk, C), lambda i: (i, 0)),
        out_shape=jax.ShapeDtypeStruct((N, C), _f32),
    )(parts, xroot, nt2, bias2, het_w, het_b)


def kernel(x, edge_index, node_type, edge_type, comp, basis, root, bias,
           het_w, het_b):
    src = edge_index[0]
    dst = edge_index[1]
    h2d, xroot = _stage_a(x, comp, basis, root)
    hflat = h2d.reshape(N * R, C)
    gidx, escale = _stage_b(src, dst, edge_type)
    parts = _stage_c(hflat, gidx, dst, escale)
    return _stage_d(parts, xroot, node_type.reshape(N, 1),
                    bias.reshape(1, C), het_w, het_b)


# speedup vs baseline: 1.0155x; 1.0155x over previous
import jax
import jax.numpy as jnp
from jax import lax
from jax.experimental import pallas as pl
from jax.experimental.pallas import tpu as pltpu
from jax.experimental.pallas import tpu_sc as plsc

N = 10000
E = 320000
C = 128
R = 8
NT = 4
NB = 4

NC = 2
NS = 16
EPT = E // (NC * NS)
RPT = N // NS
DRN = 632
CNT_PAD = 81920
TPB = CNT_PAD // NS
E_CHK = 80
N_CHK = EPT // E_CHK
E_BLK = 2000
RB = 2560

_f32 = jnp.float32
_i32 = jnp.int32
_sc_params = pltpu.CompilerParams(needs_layout_passes=False)


def _mesh():
    return plsc.VectorSubcoreMesh(core_axis_name="c", subcore_axis_name="s")


def _stage_a_body(x_ref, comp_ref, basis_ref, root_ref, h_ref, xr_ref):
    xb = x_ref[...]
    comp = comp_ref[...]
    basis = basis_ref[...]
    for r in range(R):
        w = comp[r, 0] * basis[0]
        for b in range(1, NB):
            w = w + comp[r, b] * basis[b]
        h_ref[:, r * C:(r + 1) * C] = jnp.dot(
            xb, w, preferred_element_type=_f32)
    xr_ref[...] = jnp.dot(xb, root_ref[...], preferred_element_type=_f32)


def _stage_a(x, comp, basis, root):
    blk = 2000
    return pl.pallas_call(
        _stage_a_body,
        grid=(N // blk,),
        in_specs=[
            pl.BlockSpec((blk, C), lambda i: (i, 0)),
            pl.BlockSpec((R, NB), lambda i: (0, 0)),
            pl.BlockSpec((NB, C, C), lambda i: (0, 0, 0)),
            pl.BlockSpec((C, C), lambda i: (0, 0)),
        ],
        out_specs=[
            pl.BlockSpec((blk, R * C), lambda i: (i, 0)),
            pl.BlockSpec((blk, C), lambda i: (i, 0)),
        ],
        out_shape=[
            jax.ShapeDtypeStruct((N, R * C), _f32),
            jax.ShapeDtypeStruct((N, C), _f32),
        ],
    )(x, comp, basis, root)


CROWS = CNT_PAD // C
CRPT = CROWS // NS
CCHK = 128


def _stage_b_body(zeros_hbm, src_hbm, dst_hbm, et_hbm,
                  gidx_hbm, escale_hbm,
                  cnt_loc, srcb, dstb, etb, sclb, rowidx, spcnt):
    cid = lax.axis_index("c")
    sid = lax.axis_index("s")
    base = cid * (E // NC) + sid * EPT

    pltpu.sync_copy(zeros_hbm, cnt_loc)
    pltpu.sync_copy(zeros_hbm.at[pl.ds(sid * CRPT, CRPT)],
                    spcnt.at[pl.ds(sid * CRPT, CRPT)])

    for half in range(NC):
        hbase = half * (E // NC) + sid * EPT
        pltpu.sync_copy(dst_hbm.at[pl.ds(hbase, EPT)], dstb)
        pltpu.sync_copy(et_hbm.at[pl.ds(hbase, EPT)], etb)

        def count_body(g, _):
            d16 = dstb[pl.ds(16 * g, 16)]
            t16 = etb[pl.ds(16 * g, 16)]
            cidx = d16 * R + t16
            cnts, msk = plsc.scan_count(cidx)
            plsc.addupdate_scatter(
                cnt_loc,
                [lax.shift_right_logical(cidx, 7), cidx & (C - 1)],
                cnts.astype(_f32), mask=msk)
            return 0
        lax.fori_loop(0, EPT // 16, count_body, 0)

    plsc.subcore_barrier()

    for k in range(CROWS // CCHK):
        def idx_body(i, _):
            rowidx[pl.ds(16 * i, 16)] = (
                lax.iota(_i32, 16) + (k * CCHK + 16 * i))
            return 0
        lax.fori_loop(0, CCHK // 16, idx_body, 0)
        pltpu.sync_copy(cnt_loc.at[pl.ds(k * CCHK, CCHK)],
                        spcnt.at[rowidx], add=True)

    plsc.subcore_barrier()

    pltpu.sync_copy(spcnt.at[pl.ds(sid * CRPT, CRPT)],
                    cnt_loc.at[pl.ds(0, CRPT)])
    for e in range(CRPT):
        for c in range(C // 16):
            cnt_loc[e, pl.ds(16 * c, 16)] = (
                1.0 / jnp.maximum(cnt_loc[e, pl.ds(16 * c, 16)], 1.0))
    pltpu.sync_copy(cnt_loc.at[pl.ds(0, CRPT)],
                    spcnt.at[pl.ds(sid * CRPT, CRPT)])

    plsc.subcore_barrier()
    pltpu.sync_copy(spcnt, cnt_loc)

    pltpu.sync_copy(src_hbm.at[pl.ds(base, EPT)], srcb)
    pltpu.sync_copy(dst_hbm.at[pl.ds(base, EPT)], dstb)
    pltpu.sync_copy(et_hbm.at[pl.ds(base, EPT)], etb)

    def gs_body(g, _):
        s16 = srcb[pl.ds(16 * g, 16)]
        d16 = dstb[pl.ds(16 * g, 16)]
        t16 = etb[pl.ds(16 * g, 16)]
        cidx = d16 * R + t16
        srcb[pl.ds(16 * g, 16)] = s16 * R + t16
        sclb[pl.ds(16 * g, 16)] = plsc.load_gather(
            cnt_loc,
            [lax.shift_right_logical(cidx, 7), cidx & (C - 1)])
        return 0
    lax.fori_loop(0, EPT // 16, gs_body, 0)

    pltpu.sync_copy(srcb, gidx_hbm.at[pl.ds(base, EPT)])
    pltpu.sync_copy(sclb, escale_hbm.at[pl.ds(base, EPT)])


def _stage_b(src, dst, et):
    zeros = jnp.zeros((CROWS, C), _f32)
    return pl.kernel(
        _stage_b_body,
        out_type=(jax.ShapeDtypeStruct((E,), _i32),
                  jax.ShapeDtypeStruct((E,), _f32)),
        mesh=_mesh(),
        compiler_params=_sc_params,
        scratch_types=[
            pltpu.VMEM((CROWS, C), _f32),
            pltpu.VMEM((EPT,), _i32),
            pltpu.VMEM((EPT,), _i32),
            pltpu.VMEM((EPT,), _i32),
            pltpu.VMEM((EPT,), _f32),
            pltpu.VMEM((CCHK,), _i32),
            pltpu.VMEM_SHARED((CROWS, C), _f32),
        ],
    )(zeros, src, dst, et)


def _stage_c_body(hflat_hbm, gidx_hbm, dst_hbm, escale_hbm, out2_hbm,
                  gidxb, dstb, sclb, dstw, rows0, rows1, sem0, sem1, outsp):
    cid = lax.axis_index("c")
    sid = lax.axis_index("s")
    base = cid * (E // NC) + sid * EPT
    rows_bufs = (rows0, rows1)
    sems = (sem0, sem1)

    for e in range(E_CHK):
        for c in range(C // 16):
            rows0[e, pl.ds(16 * c, 16)] = jnp.zeros((16,), _f32)

    def zero_out(k, _):
        pltpu.sync_copy(rows0, outsp.at[pl.ds(sid * RPT + k * E_CHK, E_CHK)])
        return 0
    lax.fori_loop(0, RPT // E_CHK, zero_out, 0)
    pltpu.sync_copy(rows0.at[pl.ds(0, RPT % E_CHK)],
                    outsp.at[pl.ds(sid * RPT + (RPT // E_CHK) * E_CHK,
                                   RPT % E_CHK)])

    pltpu.sync_copy(gidx_hbm.at[pl.ds(base, EPT)], gidxb)
    pltpu.sync_copy(dst_hbm.at[pl.ds(base, EPT)], dstb)
    pltpu.sync_copy(escale_hbm.at[pl.ds(base, EPT)], sclb)

    plsc.subcore_barrier()

    pltpu.async_copy(hflat_hbm.at[gidxb.at[pl.ds(0, E_CHK)]],
                     rows0, sem0)

    def process(ci, rows, sem):
        off = ci * E_CHK
        for j in range(E_CHK // 16):
            dstw[pl.ds(16 * j, 16)] = dstb[pl.ds(off + 16 * j, 16)]
        pltpu.make_async_copy(hflat_hbm.at[gidxb.at[pl.ds(off, E_CHK)]],
                              rows, sem).wait()
        for j in range(E_CHK // 16):
            sg = sclb[pl.ds(off + 16 * j, 16)]
            for el in range(16):
                e = 16 * j + el
                sv = sg[el]
                for c in range(C // 16):
                    rows[e, pl.ds(16 * c, 16)] = (
                        rows[e, pl.ds(16 * c, 16)] * sv)
        pltpu.sync_copy(rows, outsp.at[dstw], add=True)

    def chunk_body(cp, _):
        for par in range(2):
            ci = 2 * cp + par
            nxt = ci + 1
            @pl.when(nxt < N_CHK)
            def _fire():
                pltpu.async_copy(
                    hflat_hbm.at[gidxb.at[pl.ds(nxt * E_CHK, E_CHK)]],
                    rows_bufs[(par + 1) % 2], sems[(par + 1) % 2])
            process(ci, rows_bufs[par], sems[par])
        return 0
    lax.fori_loop(0, N_CHK // 2, chunk_body, 0)
    process(N_CHK - 1, rows0, sem0)

    plsc.subcore_barrier()

    @pl.when(sid < NS - 1)
    def _drain_main():
        pltpu.sync_copy(outsp.at[pl.ds(sid * DRN, DRN)],
                        out2_hbm.at[cid, pl.ds(sid * DRN, DRN)])

    @pl.when(sid == NS - 1)
    def _drain_tail():
        pltpu.sync_copy(outsp.at[pl.ds((NS - 1) * DRN, N - (NS - 1) * DRN)],
                        out2_hbm.at[cid, pl.ds((NS - 1) * DRN,
                                               N - (NS - 1) * DRN)])


def _stage_c(hflat, gidx, dst, escale):
    return pl.kernel(
        _stage_c_body,
        out_type=jax.ShapeDtypeStruct((NC, N, C), _f32),
        mesh=_mesh(),
        compiler_params=_sc_params,
        scratch_types=[
            pltpu.VMEM((EPT,), _i32),
            pltpu.VMEM((EPT,), _i32),
            pltpu.VMEM((EPT,), _f32),
            pltpu.VMEM((E_CHK,), _i32),
            pltpu.VMEM((E_CHK, C), _f32),
            pltpu.VMEM((E_CHK, C), _f32),
            pltpu.SemaphoreType.DMA,
            pltpu.SemaphoreType.DMA,
            pltpu.VMEM_SHARED((N, C), _f32),
        ],
    )(hflat, gidx, dst, escale)


def _stage_d_body(p_ref, xr_ref, nt_ref, bias_ref, hw_ref, hb_ref, out_ref):
    h = p_ref[0] + p_ref[1] + xr_ref[...] + bias_ref[...]
    h = jnp.maximum(h, 0.0)
    nt = nt_ref[...]
    acc = jnp.zeros_like(xr_ref[...])
    for t in range(NT):
        m = (nt == t).astype(_f32)
        acc = acc + jnp.dot(h * m, hw_ref[t], preferred_element_type=_f32)
        acc = acc + m * hb_ref[t:t + 1, :]
    out_ref[...] = acc


def _stage_d(parts, xroot, nt2, bias2, het_w, het_b):
    blk = 2000
    return pl.pallas_call(
        _stage_d_body,
        grid=(N // blk,),
        in_specs=[
            pl.BlockSpec((NC, blk, C), lambda i: (0, i, 0)),
            pl.BlockSpec((blk, C), lambda i: (i, 0)),
            pl.BlockSpec((blk, 1), lambda i: (i, 0)),
            pl.BlockSpec((1, C), lambda i: (0, 0)),
            pl.BlockSpec((NT, C, C), lambda i: (0, 0, 0)),
            pl.BlockSpec((NT, C), lambda i: (0, 0)),
        ],
        out_specs=pl.BlockSpec((blk, C), lambda i: (i, 0)),
        out_shape=jax.ShapeDtypeStruct((N, C), _f32),
    )(parts, xroot, nt2, bias2, het_w, het_b)


def kernel(x, edge_index, node_type, edge_type, comp, basis, root, bias,
           het_w, het_b):
    src = edge_index[0]
    dst = edge_index[1]
    h2d, xroot = _stage_a(x, comp, basis, root)
    hflat = h2d.reshape(N * R, C)
    gidx, escale = _stage_b(src, dst, edge_type)
    parts = _stage_c(hflat, gidx, dst, escale)
    return _stage_d(parts, xroot, node_type.reshape(N, 1),
                    bias.reshape(1, C), het_w, het_b)
